# baseline (speedup 1.0000x reference)
import functools

import jax
import jax.numpy as jnp
from jax import lax
from jax.experimental import pallas as pl
from jax.experimental.pallas import tpu as pltpu
from jax.experimental.pallas import tpu_sc as plsc

MAX_VAL = 52000.0
MIN_VAL = -53000.0
NUM_LEVELS = 1000
CH = 32
T = 512
D = 2048

NUM_CORES = 2
NUM_SUBCORES = 16
NW = NUM_CORES * NUM_SUBCORES
TB = T // NW
LANES = 16
VLANES = 32
CBURST = 8
DP = D // 2


def _level_indices(xr):
    clipped = jnp.minimum(jnp.maximum(xr, MIN_VAL), MAX_VAL)
    v = (NUM_LEVELS - 1) * (clipped - MIN_VAL) / (MAX_VAL - MIN_VAL)
    r0 = v.astype(jnp.int32)
    frac = v - r0.astype(jnp.float32)
    odd = jnp.bitwise_and(r0, 1)
    up = (frac > 0.5) | ((frac == 0.5) & (odd == 1))
    idx = r0 + jnp.where(up, 1, 0)
    return jnp.minimum(jnp.maximum(idx, 0), NUM_LEVELS - 1)


@functools.partial(
    pl.kernel,
    mesh=plsc.VectorSubcoreMesh(core_axis_name="c", subcore_axis_name="s"),
    out_type=jax.ShapeDtypeStruct((NW, CH, D), jnp.float32),
    scratch_types=[
        pltpu.VMEM((CH, TB), jnp.float32),
        pltpu.VMEM((CH, TB), jnp.int32),
        pltpu.VMEM((TB, D), jnp.float32),
        pltpu.VMEM((4, TB, DP), jnp.int32),
        pltpu.VMEM((CBURST, D), jnp.float32),
        pltpu.SemaphoreType.DMA,
        pltpu.SemaphoreType.DMA,
        pltpu.SemaphoreType.DMA,
        pltpu.SemaphoreType.DMA,
        pltpu.SemaphoreType.DMA,
        pltpu.SemaphoreType.DMA,
    ],
)
def _sc_encode(xf_hbm, level_hbm, feat_hbm, part_hbm,
               x_v, idx_v, feat_v, gbuf, obuf,
               gsem0, gsem1, gsem2, gsem3, osem, xsem):
    wid = lax.axis_index("s") * NUM_CORES + lax.axis_index("c")
    t0 = wid * TB
    gsems = (gsem0, gsem1, gsem2, gsem3)
    NB = 4

    def xcopy(c):
        return pltpu.make_async_copy(
            xf_hbm.at[pl.ds(c * T + t0, TB)], x_v.at[c], xsem)

    for c in range(CH):
        xcopy(c).start()
    pltpu.sync_copy(feat_hbm.at[pl.ds(t0, TB), :], feat_v)
    for c in range(CH):
        xcopy(c).wait()

    for c in range(CH):
        idx_v[c, :] = _level_indices(x_v[c, :])

    def gcopy(c):
        return pltpu.make_async_copy(
            level_hbm.at[idx_v.at[c]], gbuf.at[c % NB], gsems[c % NB])

    def ocopy(c0):
        return pltpu.make_async_copy(
            obuf, part_hbm.at[wid, pl.ds(c0, CBURST)], osem)

    MSK = jnp.int32(-65536)
    f32 = jnp.float32
    bc = lambda v: lax.bitcast_convert_type(v, f32)

    for c in range(4):
        gcopy(c).start()
    for k in range(CH // 2):
        c0 = 2 * k
        b0, b1 = c0 % NB, (c0 + 1) % NB
        co0, co1 = c0 % CBURST, (c0 + 1) % CBURST
        if c0 % CBURST == 0 and c0 >= CBURST:
            ocopy(c0 - CBURST).wait()
        gcopy(c0).wait()
        gcopy(c0 + 1).wait()

        def dbody(i, _, b0=b0, b1=b1, co0=co0, co1=co1):
            slw = pl.ds(i * LANES, LANES)
            sll = pl.ds(i * LANES, LANES)
            slh = pl.ds(DP + i * LANES, LANES)
            acc = [None] * 8
            for tt in range(TB):
                w0 = gbuf[b0, tt, slw]
                w1 = gbuf[b1, tt, slw]
                flo = feat_v[tt, sll]
                fhi = feat_v[tt, slh]
                terms = (bc(w0 << 16) * flo, bc(w0 & MSK) * fhi,
                         bc(w1 << 16) * flo, bc(w1 & MSK) * fhi)
                k = 4 * (tt & 1)
                for j, t in enumerate(terms):
                    a = acc[k + j]
                    acc[k + j] = t if a is None else a + t
            obuf[co0, sll] = acc[0] + acc[4]
            obuf[co0, slh] = acc[1] + acc[5]
            obuf[co1, sll] = acc[2] + acc[6]
            obuf[co1, slh] = acc[3] + acc[7]
            return 0

        lax.fori_loop(0, DP // LANES, dbody, 0)
        if c0 + 4 < CH:
            gcopy(c0 + 4).start()
            gcopy(c0 + 5).start()
        if (c0 + 1) % CBURST == CBURST - 1:
            ocopy(c0 + 1 - (CBURST - 1)).start()
    ocopy(CH - CBURST).wait()


def _tc_pack_body(x_ref, o_ref):
    b = lax.bitcast_convert_type(x_ref[...], jnp.int32)
    o_ref[...] = (lax.shift_right_logical(b[:, :DP], 16)
                  | (b[:, DP:] & jnp.int32(-65536)))


def _tc_finish_body(part_ref, ch_ref, out_ref):
    s = jnp.sum(part_ref[...], axis=0)
    s = jnp.where(s > 0, 1.0, -1.0).astype(jnp.float32)
    bnd = s * ch_ref[...]

    def rolled(v, k):
        return jnp.concatenate([v[:, D - k:], v[:, :D - k]], axis=1)

    ng = (rolled(bnd[0:CH - 3], 3) * rolled(bnd[1:CH - 2], 2)
          * rolled(bnd[2:CH - 1], 1) * bnd[3:CH])
    o = jnp.sum(ng, axis=0, keepdims=True)
    out_ref[...] = jnp.where(o > 0, 1.0, -1.0).astype(jnp.float32)


def kernel(input, level_w, feat_w, ch_w):
    level_packed = pl.pallas_call(
        _tc_pack_body,
        grid=(NUM_LEVELS // 200,),
        in_specs=[pl.BlockSpec((200, D), lambda i: (i, 0))],
        out_specs=pl.BlockSpec((200, DP), lambda i: (i, 0)),
        out_shape=jax.ShapeDtypeStruct((NUM_LEVELS, DP), jnp.int32),
    )(level_w)
    partials = _sc_encode(input.reshape(-1), level_packed, feat_w)
    return pl.pallas_call(
        _tc_finish_body,
        out_shape=jax.ShapeDtypeStruct((1, D), jnp.float32),
    )(partials, ch_w)


# speedup vs baseline: 1.1670x; 1.1670x over previous
import functools

import jax
import jax.numpy as jnp
from jax import lax
from jax.experimental import pallas as pl
from jax.experimental.pallas import tpu as pltpu
from jax.experimental.pallas import tpu_sc as plsc

MAX_VAL = 52000.0
MIN_VAL = -53000.0
NUM_LEVELS = 1000
CH = 32
T = 512
D = 2048

NUM_CORES = 2
NUM_SUBCORES = 16
NW = NUM_CORES * NUM_SUBCORES
TB = T // NW
LANES = 16
VLANES = 32
CBURST = 8
DP = D // 2


def _level_indices(xr):
    clipped = jnp.minimum(jnp.maximum(xr, MIN_VAL), MAX_VAL)
    v = (NUM_LEVELS - 1) * (clipped - MIN_VAL) / (MAX_VAL - MIN_VAL)
    r0 = v.astype(jnp.int32)
    frac = v - r0.astype(jnp.float32)
    odd = jnp.bitwise_and(r0, 1)
    up = (frac > 0.5) | ((frac == 0.5) & (odd == 1))
    idx = r0 + jnp.where(up, 1, 0)
    return jnp.minimum(jnp.maximum(idx, 0), NUM_LEVELS - 1)


@functools.partial(
    pl.kernel,
    mesh=plsc.VectorSubcoreMesh(core_axis_name="c", subcore_axis_name="s"),
    out_type=jax.ShapeDtypeStruct((NW, CH, D), jnp.float32),
    scratch_types=[
        pltpu.VMEM((CH, TB), jnp.float32),
        pltpu.VMEM((CH, TB), jnp.int32),
        pltpu.VMEM((TB, D), jnp.float32),
        pltpu.VMEM((4, TB, DP), jnp.int32),
        pltpu.VMEM((CBURST, D), jnp.float32),
        pltpu.SemaphoreType.DMA,
        pltpu.SemaphoreType.DMA,
        pltpu.SemaphoreType.DMA,
        pltpu.SemaphoreType.DMA,
        pltpu.SemaphoreType.DMA,
        pltpu.SemaphoreType.DMA,
    ],
)
def _sc_encode(xf_hbm, level_hbm, feat_hbm, part_hbm,
               x_v, idx_v, feat_v, gbuf, obuf,
               gsem0, gsem1, gsem2, gsem3, osem, xsem):
    wid = lax.axis_index("s") * NUM_CORES + lax.axis_index("c")
    t0 = wid * TB
    gsems = (gsem0, gsem1, gsem2, gsem3)
    NB = 4

    def xcopy(c):
        return pltpu.make_async_copy(
            xf_hbm.at[pl.ds(c * T + t0, TB)], x_v.at[c], xsem)

    for c in range(CH):
        xcopy(c).start()
    pltpu.sync_copy(feat_hbm.at[pl.ds(t0, TB), :], feat_v)
    for c in range(CH):
        xcopy(c).wait()

    for c in range(CH):
        idx_v[c, :] = _level_indices(x_v[c, :])

    def gcopy(c):
        return pltpu.make_async_copy(
            level_hbm.at[idx_v.at[c]], gbuf.at[c % NB], gsems[c % NB])

    def ocopy(c0):
        return pltpu.make_async_copy(
            obuf, part_hbm.at[wid, pl.ds(c0, CBURST)], osem)

    MSK = jnp.int32(-65536)
    f32 = jnp.float32
    bc = lambda v: lax.bitcast_convert_type(v, f32)

    for c in range(4):
        gcopy(c).start()
    for k in range(CH // 2):
        c0 = 2 * k
        b0, b1 = c0 % NB, (c0 + 1) % NB
        co0, co1 = c0 % CBURST, (c0 + 1) % CBURST
        if c0 % CBURST == 0 and c0 >= CBURST:
            ocopy(c0 - CBURST).wait()
        gcopy(c0).wait()
        gcopy(c0 + 1).wait()

        @plsc.parallel_loop(0, DP // LANES, 1)
        def dbody(i, b0=b0, b1=b1, co0=co0, co1=co1):
            slw = pl.ds(i * LANES, LANES)
            sll = pl.ds(i * LANES, LANES)
            slh = pl.ds(DP + i * LANES, LANES)
            w0 = gbuf[b0, 0, slw]
            w1 = gbuf[b1, 0, slw]
            flo = feat_v[0, sll]
            fhi = feat_v[0, slh]
            a0l = bc(w0 << 16) * flo
            a0h = bc(w0 & MSK) * fhi
            a1l = bc(w1 << 16) * flo
            a1h = bc(w1 & MSK) * fhi
            for tt in range(1, TB):
                w0 = gbuf[b0, tt, slw]
                w1 = gbuf[b1, tt, slw]
                flo = feat_v[tt, sll]
                fhi = feat_v[tt, slh]
                a0l = a0l + bc(w0 << 16) * flo
                a0h = a0h + bc(w0 & MSK) * fhi
                a1l = a1l + bc(w1 << 16) * flo
                a1h = a1h + bc(w1 & MSK) * fhi
            obuf[co0, sll] = a0l
            obuf[co0, slh] = a0h
            obuf[co1, sll] = a1l
            obuf[co1, slh] = a1h
        if c0 + 4 < CH:
            gcopy(c0 + 4).start()
            gcopy(c0 + 5).start()
        if (c0 + 1) % CBURST == CBURST - 1:
            ocopy(c0 + 1 - (CBURST - 1)).start()
    ocopy(CH - CBURST).wait()


def _tc_pack_body(x_ref, o_ref):
    b = lax.bitcast_convert_type(x_ref[...], jnp.int32)
    o_ref[...] = (lax.shift_right_logical(b[:, :DP], 16)
                  | (b[:, DP:] & jnp.int32(-65536)))


def _tc_finish_body(part_ref, ch_ref, out_ref):
    s = jnp.sum(part_ref[...], axis=0)
    s = jnp.where(s > 0, 1.0, -1.0).astype(jnp.float32)
    bnd = s * ch_ref[...]

    def rolled(v, k):
        return jnp.concatenate([v[:, D - k:], v[:, :D - k]], axis=1)

    ng = (rolled(bnd[0:CH - 3], 3) * rolled(bnd[1:CH - 2], 2)
          * rolled(bnd[2:CH - 1], 1) * bnd[3:CH])
    o = jnp.sum(ng, axis=0, keepdims=True)
    out_ref[...] = jnp.where(o > 0, 1.0, -1.0).astype(jnp.float32)


def kernel(input, level_w, feat_w, ch_w):
    level_packed = pl.pallas_call(
        _tc_pack_body,
        grid=(NUM_LEVELS // 200,),
        in_specs=[pl.BlockSpec((200, D), lambda i: (i, 0))],
        out_specs=pl.BlockSpec((200, DP), lambda i: (i, 0)),
        out_shape=jax.ShapeDtypeStruct((NUM_LEVELS, DP), jnp.int32),
    )(level_w)
    partials = _sc_encode(input.reshape(-1), level_packed, feat_w)
    return pl.pallas_call(
        _tc_finish_body,
        out_shape=jax.ShapeDtypeStruct((1, D), jnp.float32),
    )(partials, ch_w)
